# baseline (speedup 1.0000x reference)
import functools
import math

import jax
import jax.numpy as jnp
from jax.experimental import pallas as pl
from jax.experimental.pallas import tpu as pltpu

B = 64
C = 3
IMG = 224
P = 16
D = 384
H = 8
L = 6
FF = 1536
NPATCH = (IMG // P) ** 2
N = NPATCH + 1
NP = 208
E = 8
TOPK = 2
ED = 256
KEEP = int(NPATCH * 0.5)
KP = 104
HD = D // H

PREC = jax.lax.Precision.DEFAULT
NEG = -1e30


def _mm(a, b, prec=PREC):
    return jax.lax.dot_general(a, b, (((1,), (0,)), ((), ())),
                               precision=prec,
                               preferred_element_type=jnp.float32)


def _mm_t(a, b, prec=PREC):
    return jax.lax.dot_general(a, b, (((1,), (1,)), ((), ())),
                               precision=prec,
                               preferred_element_type=jnp.float32)


def _xsum(x):
    rows, w = x.shape
    nv = -(-w // 128)
    if w % 128 != 0:
        x = jnp.concatenate(
            [x, jnp.zeros((rows, nv * 128 - w), jnp.float32)], axis=1)
    s = x[:, :128]
    for i in range(1, nv):
        s = s + x[:, i * 128:(i + 1) * 128]
    t = s + pltpu.roll(s, 127, 1)
    t = t + pltpu.roll(t, 126, 1)
    t = t + pltpu.roll(t, 124, 1)
    acc = t[:, 0:8]
    for j in range(1, 16):
        acc = acc + t[:, 8 * j:8 * j + 8]
    return acc[:, 0:1]


def _softmax(s):
    m = jnp.max(s, axis=-1, keepdims=True)
    es = jnp.exp(s - m)
    return es / _xsum(es)


def _ln(x, w, b, eps=1e-5):
    n = x.shape[-1]
    mu = _xsum(x) * (1.0 / n)
    var = _xsum((x - mu) ** 2) * (1.0 / n)
    return (x - mu) / jnp.sqrt(var + eps) * w + b


def _gelu(x):
    return 0.5 * x * (1.0 + jax.lax.erf(x * (1.0 / math.sqrt(2.0))))


def _embed_kernel(xp_ref, w_ref, cb_ref, pos_ref, out_ref):
    xp = xp_ref[0]
    t = _mm_t(xp, w_ref[...])
    out_ref[0] = t + cb_ref[...] + pos_ref[...]


def _layer_kernel(h_ref, wqkv_ref, bqkv_ref, wout_ref, bout_ref,
                  ln1w_ref, ln1b_ref, w1_ref, b1_ref, w2_ref, b2_ref,
                  ln2w_ref, ln2b_ref, out_ref):
    h = h_ref[0]
    qkv = _mm_t(h, wqkv_ref[...]) + bqkv_ref[...]

    key_idx = jax.lax.broadcasted_iota(jnp.int32, (NP, NP), 1)
    key_mask = key_idx < N

    outs = []
    for hh in range(H):
        q = qkv[:, hh * HD:(hh + 1) * HD]
        k = qkv[:, D + hh * HD:D + (hh + 1) * HD]
        v = qkv[:, 2 * D + hh * HD:2 * D + (hh + 1) * HD]
        s = _mm_t(q, k) / math.sqrt(HD)
        s = jnp.where(key_mask, s, NEG)
        p = _softmax(s)
        outs.append(_mm(p, v))
    o = jnp.concatenate(outs, axis=1)
    o = _mm_t(o, wout_ref[...]) + bout_ref[...]

    h1 = _ln(h + o, ln1w_ref[...], ln1b_ref[...])
    ff = jnp.maximum(_mm_t(h1, w1_ref[...]) + b1_ref[...], 0.0)
    ff = _mm_t(ff, w2_ref[...]) + b2_ref[...]
    out_ref[0] = _ln(h1 + ff, ln2w_ref[...], ln2b_ref[...])


def _route_kernel(hp_ref, sw1_ref, sb1_ref, sw2_ref, gw_ref, gb_ref,
                  sel_ref, gate_ref):
    hp = hp_ref[0]
    s1 = jnp.maximum(_mm_t(hp, sw1_ref[...]) + sb1_ref[...], 0.0)
    col = _xsum(s1 * sw2_ref[...])
    row = col.T

    ii = jax.lax.broadcasted_iota(jnp.int32, (NPATCH, NPATCH), 0)
    jj = jax.lax.broadcasted_iota(jnp.int32, (NPATCH, NPATCH), 1)
    beats = (row > col) | ((row == col) & (jj < ii))
    rank = jnp.sum(beats.astype(jnp.float32), axis=1, keepdims=True)

    rr = jax.lax.broadcasted_iota(jnp.int32, (KP, NPATCH), 0).astype(jnp.float32)
    M = (rank.T == rr).astype(jnp.float32)
    sel = _mm(M, hp)
    sel_ref[0] = sel

    gl = _mm_t(sel, gw_ref[...]) + gb_ref[...]
    eio = jax.lax.broadcasted_iota(jnp.int32, (KP, E), 1)
    m1 = jnp.max(gl, axis=1, keepdims=True)
    i1 = jnp.min(jnp.where(gl == m1, eio, E), axis=1, keepdims=True)
    sel1 = eio == i1
    glm = jnp.where(sel1, NEG, gl)
    m2 = jnp.max(glm, axis=1, keepdims=True)
    i2 = jnp.min(jnp.where(glm == m2, eio, E), axis=1, keepdims=True)
    sel2 = eio == i2
    eb = jnp.exp(m2 - m1)
    g1 = 1.0 / (1.0 + eb)
    g2 = eb / (1.0 + eb)
    keep = jax.lax.broadcasted_iota(jnp.int32, (KP, E), 0) < KEEP
    gates = jnp.where(sel1, g1, 0.0) + jnp.where(sel2, g2, 0.0)
    gate_ref[0] = jnp.where(keep, gates, 0.0)


def _expert_kernel(sel_ref, gate_ref, ew1_ref, eb1_ref, ew2_ref, eb2_ref,
                   ew3_ref, eb3_ref, out_ref):
    xs = sel_ref[0]
    g = gate_ref[0]
    acc = jnp.zeros((KP, ED), jnp.float32)
    for e in range(E):
        h1 = _gelu(_mm_t(xs, ew1_ref[e]) + eb1_ref[e:e + 1, :])
        h2 = _gelu(_mm_t(h1, ew2_ref[e]) + eb2_ref[e:e + 1, :])
        oe = _mm_t(h2, ew3_ref[e]) + eb3_ref[e:e + 1, :]
        acc = acc + oe * g[:, e:e + 1]
    out_ref[0] = jnp.sum(acc, axis=0, keepdims=True)


def _full(shape):
    return pl.BlockSpec(shape, lambda b: (0,) * len(shape))


def _batch(shape):
    return pl.BlockSpec(shape, lambda b: (b,) + (0,) * (len(shape) - 1))


def kernel(x, conv_w, conv_b, cls_token, pos_embed, in_proj_w, in_proj_b,
           out_proj_w, out_proj_b, lin1_w, lin1_b, lin2_w, lin2_b,
           ln1_w, ln1_b, ln2_w, ln2_b, score_w1, score_b1, score_w2, score_b2,
           gate_w, gate_b, ew1, eb1, ew2, eb2, ew3, eb3):
    f32 = jnp.float32
    xp = x.reshape(B, C, IMG // P, P, IMG // P, P).transpose(0, 2, 4, 1, 3, 5)
    xp = xp.reshape(B, NPATCH, C * P * P)
    wmat = conv_w.reshape(D, C * P * P)

    tokens = pl.pallas_call(
        _embed_kernel,
        grid=(B,),
        in_specs=[_batch((1, NPATCH, C * P * P)), _full((D, C * P * P)),
                  _full((1, D)), _full((NPATCH, D))],
        out_specs=_batch((1, NPATCH, D)),
        out_shape=jax.ShapeDtypeStruct((B, NPATCH, D), f32),
    )(xp, wmat, conv_b.reshape(1, D), pos_embed[0, 1:, :])

    cls_row = jnp.broadcast_to((cls_token[0, 0] + pos_embed[0, 0]).reshape(1, 1, D),
                               (B, 1, D))
    h = jnp.concatenate(
        [cls_row, tokens, jnp.zeros((B, NP - N, D), f32)], axis=1)

    layer_call = pl.pallas_call(
        _layer_kernel,
        grid=(B,),
        in_specs=[_batch((1, NP, D)), _full((3 * D, D)), _full((1, 3 * D)),
                  _full((D, D)), _full((1, D)), _full((1, D)), _full((1, D)),
                  _full((FF, D)), _full((1, FF)), _full((D, FF)), _full((1, D)),
                  _full((1, D)), _full((1, D))],
        out_specs=_batch((1, NP, D)),
        out_shape=jax.ShapeDtypeStruct((B, NP, D), f32),
    )
    for i in range(L):
        h = layer_call(h, in_proj_w[i], in_proj_b[i].reshape(1, 3 * D),
                       out_proj_w[i], out_proj_b[i].reshape(1, D),
                       ln1_w[i].reshape(1, D), ln1_b[i].reshape(1, D),
                       lin1_w[i], lin1_b[i].reshape(1, FF),
                       lin2_w[i], lin2_b[i].reshape(1, D),
                       ln2_w[i].reshape(1, D), ln2_b[i].reshape(1, D))

    patch = h[:, 1:N, :]

    selected, gates = pl.pallas_call(
        _route_kernel,
        grid=(B,),
        in_specs=[_batch((1, NPATCH, D)), _full((D // 2, D)), _full((1, D // 2)),
                  _full((1, D // 2)), _full((E, D)), _full((1, E))],
        out_specs=[_batch((1, KP, D)), _batch((1, KP, E))],
        out_shape=[jax.ShapeDtypeStruct((B, KP, D), f32),
                   jax.ShapeDtypeStruct((B, KP, E), f32)],
    )(patch, score_w1, score_b1.reshape(1, D // 2), score_w2, gate_w,
      gate_b.reshape(1, E))

    final = pl.pallas_call(
        _expert_kernel,
        grid=(B,),
        in_specs=[_batch((1, KP, D)), _batch((1, KP, E)),
                  _full((E, ED, D)), _full((E, ED)),
                  _full((E, ED, ED)), _full((E, ED)),
                  _full((E, ED, ED)), _full((E, ED))],
        out_specs=_batch((1, 1, ED)),
        out_shape=jax.ShapeDtypeStruct((B, 1, ED), f32),
    )(selected, gates, ew1, eb1, ew2, eb2, ew3, eb3)

    return final.reshape(B, ED)
